# baseline (speedup 1.0000x reference)
import jax
import jax.numpy as jnp
from jax.experimental import pallas as pl
from jax.experimental.pallas import tpu as pltpu


def _bins(n):
    return [((i * n) // 3, ((i + 1) * n + 2) // 3) for i in range(3)]


def _make_body(B, CB, H, W, eps, rc):
    Hh, W2 = H // 2, 2 * W
    inv_n = 1.0 / float(B * H * W)
    hb, wb = _bins(H), _bins(W)

    def body(x_ref, g_ref, b_ref, y_ref):
        f32 = jnp.float32
        u = jax.lax.broadcasted_iota(jnp.int32, (1, 1, 1, W2), 3)
        w_idx = u % W
        lo_b = u < W
        lo_m = lo_b.astype(f32)
        hi_m = 1.0 - lo_m

        def roll64(v):
            return jnp.concatenate([v[..., W:], v[..., :W]], axis=-1)

        s_top = jnp.where(lo_b, 0.0,
                          roll64(x_ref[:, :, Hh - 1:Hh, :].astype(f32)))
        s_bot = jnp.where(lo_b,
                          roll64(x_ref[:, :, 0:1, :].astype(f32)), 0.0)

        taps = []
        for (hs, he) in hb:
            parts = []
            if hs < Hh:
                s = jnp.sum(x_ref[:, :, hs:min(he, Hh), :].astype(f32),
                            axis=2, keepdims=True)
                parts.append((s, lo_m))
            if he > Hh:
                s = jnp.sum(
                    x_ref[:, :, max(hs, Hh) - Hh:he - Hh, :].astype(f32),
                    axis=2, keepdims=True)
                parts.append((s, hi_m))
            row = []
            for (ws, we) in wb:
                wm = ((w_idx >= ws) & (w_idx < we)).astype(f32)
                t = None
                for s, hm in parts:
                    term = jnp.sum(s * (wm * hm), axis=3, keepdims=True)
                    t = term if t is None else t + term
                row.append(t * (1.0 / float((he - hs) * (we - ws))))
            taps.append(row)

        s1v = jnp.zeros((B, CB, 1, W2), f32)
        s2v = jnp.zeros((B, CB, 1, W2), f32)
        for r0 in range(0, Hh, rc):
            rcs = min(rc, Hh - r0)
            cen = x_ref[:, :, r0:r0 + rcs, :].astype(f32)
            if r0 == 0:
                up_in = x_ref[:, :, 0:r0 + rcs - 1, :].astype(f32)
                xup = jnp.concatenate([s_top, up_in], axis=2)
            else:
                xup = x_ref[:, :, r0 - 1:r0 + rcs - 1, :].astype(f32)
            if r0 + rcs == Hh:
                dn_in = x_ref[:, :, r0 + 1:Hh, :].astype(f32)
                xdn = jnp.concatenate([dn_in, s_bot], axis=2)
            else:
                xdn = x_ref[:, :, r0 + 1:r0 + rcs + 1, :].astype(f32)
            acc = None
            for ki, v in ((0, xup), (1, cen), (2, xdn)):
                vl = jnp.where(w_idx == W - 1, 0.0,
                               jnp.concatenate([v[..., 1:], v[..., :1]], -1))
                vr = jnp.where(w_idx == 0, 0.0,
                               jnp.concatenate([v[..., -1:], v[..., :-1]], -1))
                part = taps[ki][0] * vr + taps[ki][1] * v + taps[ki][2] * vl
                acc = part if acc is None else acc + part
            s1v = s1v + jnp.sum(acc, axis=2, keepdims=True)
            s2v = s2v + jnp.sum(acc * acc, axis=2, keepdims=True)
            y_ref[:, :, r0:r0 + rcs, :] = acc.astype(y_ref.dtype)

        s1 = jnp.sum(jnp.sum(s1v, axis=3, keepdims=True), axis=0,
                     keepdims=True)
        s2 = jnp.sum(jnp.sum(s2v, axis=3, keepdims=True), axis=0,
                     keepdims=True)
        mean = s1 * inv_n
        var = s2 * inv_n - mean * mean
        g = g_ref[...].astype(f32).reshape(1, CB, 1, W2)
        b = b_ref[...].astype(f32).reshape(1, CB, 1, W2)
        scale = g * jax.lax.rsqrt(var + eps)
        bias = b - mean * scale

        for r0 in range(0, Hh, rc):
            rcs = min(rc, Hh - r0)
            yv = y_ref[:, :, r0:r0 + rcs, :].astype(f32)
            y_ref[:, :, r0:r0 + rcs, :] = jnp.maximum(
                yv * scale + bias, 0.0).astype(y_ref.dtype)

    return body


def _dcm(x, gamma, beta, cb=1, rc=8, eps=1e-5):
    B, C, H, W = x.shape
    assert H % 2 == 0 and C % cb == 0
    Hh, W2 = H // 2, 2 * W
    xd = jnp.concatenate([x[:, :, :Hh, :], x[:, :, Hh:, :]], axis=-1)
    gl = jnp.broadcast_to(gamma.astype(jnp.float32).reshape(C, 1, 1),
                          (C, 1, W2))
    bl = jnp.broadcast_to(beta.astype(jnp.float32).reshape(C, 1, 1),
                          (C, 1, W2))
    body = _make_body(B, cb, H, W, float(eps), rc)
    yd = pl.pallas_call(
        body,
        out_shape=jax.ShapeDtypeStruct((B, C, Hh, W2), x.dtype),
        grid=(C // cb,),
        in_specs=[
            pl.BlockSpec((B, cb, Hh, W2), lambda c: (0, c, 0, 0)),
            pl.BlockSpec((cb, 1, W2), lambda c: (c, 0, 0)),
            pl.BlockSpec((cb, 1, W2), lambda c: (c, 0, 0)),
        ],
        out_specs=pl.BlockSpec((B, cb, Hh, W2), lambda c: (0, c, 0, 0)),
        compiler_params=pltpu.CompilerParams(
            dimension_semantics=("parallel",),
            vmem_limit_bytes=48 << 20),
    )(xd, gl, bl)
    return jnp.concatenate([yd[..., :W], yd[..., W:]], axis=2)


def kernel(x, gamma, beta):
    return _dcm(x, gamma, beta)


# speedup vs baseline: 1.6867x; 1.6867x over previous
import jax
import jax.numpy as jnp
from jax.experimental import pallas as pl
from jax.experimental.pallas import tpu as pltpu


def _bins(n):
    return [((i * n) // 3, ((i + 1) * n + 2) // 3) for i in range(3)]


def _row_bin_sum(x_ref, hs, he, lo_m, hi_m, f32):
    fs, fe = (hs + 1) // 2, he // 2
    terms = []
    if fe > fs:
        terms.append(jnp.sum(x_ref[:, :, fs:fe, :].astype(f32), axis=2,
                             keepdims=True))
    if hs % 2 == 1:
        terms.append(x_ref[:, :, hs // 2:hs // 2 + 1, :].astype(f32) * hi_m)
    if he % 2 == 1:
        terms.append(x_ref[:, :, he // 2:he // 2 + 1, :].astype(f32) * lo_m)
    out = terms[0]
    for t in terms[1:]:
        out = out + t
    return out


def _make_body(B, CB, H, W, eps, rc):
    Hh, W2 = H // 2, 2 * W
    inv_n = 1.0 / float(B * H * W)
    hb, wb = _bins(H), _bins(W)

    def body(x_ref, g_ref, b_ref, y_ref):
        f32 = jnp.float32
        u = jax.lax.broadcasted_iota(jnp.int32, (1, 1, 1, W2), 3)
        w_idx = u % W
        lo_b = u < W
        lo_m = lo_b.astype(f32)
        hi_m = 1.0 - lo_m

        def roll64(v):
            return jnp.concatenate([v[..., W:], v[..., :W]], axis=-1)

        taps = []
        for (hs, he) in hb:
            srow = _row_bin_sum(x_ref, hs, he, lo_m, hi_m, f32)
            row = []
            for (ws, we) in wb:
                wm = ((w_idx >= ws) & (w_idx < we)).astype(f32)
                t = jnp.sum(srow * wm, axis=3, keepdims=True)
                row.append(t * (1.0 / float((he - hs) * (we - ws))))
            taps.append(row)

        zrow = jnp.zeros((B, CB, 1, W2), f32)
        s1v = zrow
        s2v = zrow
        for r0 in range(0, Hh, rc):
            rcs = min(rc, Hh - r0)
            cen = x_ref[:, :, r0:r0 + rcs, :].astype(f32)
            a = max(r0 - 1, 0)
            b_ = min(r0 + rcs + 1, Hh)
            rz = roll64(x_ref[:, :, a:b_, :].astype(f32))
            if r0 == 0:
                rz = jnp.concatenate([zrow, rz], axis=2)
            if r0 + rcs == Hh:
                rz = jnp.concatenate([rz, zrow], axis=2)
            xup = jnp.where(lo_b, rz[:, :, 0:rcs, :], rz[:, :, 1:rcs + 1, :])
            xdn = jnp.where(lo_b, rz[:, :, 1:rcs + 1, :],
                            rz[:, :, 2:rcs + 2, :])
            acc = None
            for ki, v in ((0, xup), (1, cen), (2, xdn)):
                vl = jnp.where(w_idx == W - 1, 0.0,
                               jnp.concatenate([v[..., 1:], v[..., :1]], -1))
                vr = jnp.where(w_idx == 0, 0.0,
                               jnp.concatenate([v[..., -1:], v[..., :-1]], -1))
                part = taps[ki][0] * vr + taps[ki][1] * v + taps[ki][2] * vl
                acc = part if acc is None else acc + part
            s1v = s1v + jnp.sum(acc, axis=2, keepdims=True)
            s2v = s2v + jnp.sum(acc * acc, axis=2, keepdims=True)
            y_ref[:, :, r0:r0 + rcs, :] = acc.astype(y_ref.dtype)

        s1 = jnp.sum(jnp.sum(s1v, axis=3, keepdims=True), axis=0,
                     keepdims=True)
        s2 = jnp.sum(jnp.sum(s2v, axis=3, keepdims=True), axis=0,
                     keepdims=True)
        mean = s1 * inv_n
        var = s2 * inv_n - mean * mean
        g = g_ref[...].astype(f32).reshape(1, CB, 1, W2)
        b = b_ref[...].astype(f32).reshape(1, CB, 1, W2)
        scale = g * jax.lax.rsqrt(var + eps)
        bias = b - mean * scale

        for r0 in range(0, Hh, rc):
            rcs = min(rc, Hh - r0)
            yv = y_ref[:, :, r0:r0 + rcs, :].astype(f32)
            y_ref[:, :, r0:r0 + rcs, :] = jnp.maximum(
                yv * scale + bias, 0.0).astype(y_ref.dtype)

    return body


def _dcm(x, gamma, beta, cb=1, rc=8, eps=1e-5):
    B, C, H, W = x.shape
    assert H % 2 == 0 and C % cb == 0
    Hh, W2 = H // 2, 2 * W
    xd = x.reshape(B, C, Hh, W2)
    gl = jnp.broadcast_to(gamma.astype(jnp.float32).reshape(C, 1, 1),
                          (C, 1, W2))
    bl = jnp.broadcast_to(beta.astype(jnp.float32).reshape(C, 1, 1),
                          (C, 1, W2))
    body = _make_body(B, cb, H, W, float(eps), rc)
    yd = pl.pallas_call(
        body,
        out_shape=jax.ShapeDtypeStruct((B, C, Hh, W2), x.dtype),
        grid=(C // cb,),
        in_specs=[
            pl.BlockSpec((B, cb, Hh, W2), lambda c: (0, c, 0, 0)),
            pl.BlockSpec((cb, 1, W2), lambda c: (c, 0, 0)),
            pl.BlockSpec((cb, 1, W2), lambda c: (c, 0, 0)),
        ],
        out_specs=pl.BlockSpec((B, cb, Hh, W2), lambda c: (0, c, 0, 0)),
        compiler_params=pltpu.CompilerParams(
            dimension_semantics=("parallel",),
            vmem_limit_bytes=48 << 20),
    )(xd, gl, bl)
    return yd.reshape(B, C, H, W)


def kernel(x, gamma, beta):
    return _dcm(x, gamma, beta)


# speedup vs baseline: 2.0850x; 1.2361x over previous
import jax
import jax.numpy as jnp
from jax.experimental import pallas as pl
from jax.experimental.pallas import tpu as pltpu


def _bins(n):
    return [((i * n) // 3, ((i + 1) * n + 2) // 3) for i in range(3)]


def _row_bin_sum(x_ref, hs, he, lo_m, hi_m, f32):
    fs, fe = (hs + 1) // 2, he // 2
    terms = []
    if fe > fs:
        terms.append(jnp.sum(x_ref[:, :, fs:fe, :].astype(f32), axis=2,
                             keepdims=True))
    if hs % 2 == 1:
        terms.append(x_ref[:, :, hs // 2:hs // 2 + 1, :].astype(f32) * hi_m)
    if he % 2 == 1:
        terms.append(x_ref[:, :, he // 2:he // 2 + 1, :].astype(f32) * lo_m)
    out = terms[0]
    for t in terms[1:]:
        out = out + t
    return out


def _make_body(B, CB, H, W, eps, rc):
    Hh, W2 = H // 2, 2 * W
    inv_n = 1.0 / float(B * H * W)
    hb, wb = _bins(H), _bins(W)

    def body(x_ref, g_ref, b_ref, y_ref, rz_s):
        f32 = jnp.float32
        u = jax.lax.broadcasted_iota(jnp.int32, (1, 1, 1, W2), 3)
        w_idx = u % W
        lo_b = u < W
        lo_m = lo_b.astype(f32)
        hi_m = 1.0 - lo_m

        z = x_ref[...].astype(f32)
        rz = jnp.concatenate([z[..., W:], z[..., :W]], axis=-1)
        rz_s[:, :, 1:Hh + 1, :] = rz
        zrow = jnp.zeros((B, CB, 1, W2), f32)
        rz_s[:, :, 0:1, :] = zrow
        rz_s[:, :, Hh + 1:Hh + 2, :] = zrow

        taps = []
        for (hs, he) in hb:
            srow = _row_bin_sum(x_ref, hs, he, lo_m, hi_m, f32)
            row = []
            for (ws, we) in wb:
                wm = ((w_idx >= ws) & (w_idx < we)).astype(f32)
                t = jnp.sum(srow * wm, axis=3, keepdims=True)
                row.append(t * (1.0 / float((he - hs) * (we - ws))))
            taps.append(row)
        m_notw0 = (w_idx != 0).astype(f32)
        m_notwl = (w_idx != W - 1).astype(f32)
        tr = [taps[ki][0] * m_notw0 for ki in range(3)]
        tc = [taps[ki][1] for ki in range(3)]
        tl = [taps[ki][2] * m_notwl for ki in range(3)]

        s1v = jnp.zeros((B, CB, 1, W2), f32)
        s2v = jnp.zeros((B, CB, 1, W2), f32)
        for r0 in range(0, Hh, rc):
            rcs = min(rc, Hh - r0)
            cen = x_ref[:, :, r0:r0 + rcs, :].astype(f32)
            p0 = rz_s[:, :, r0:r0 + rcs, :]
            p1 = rz_s[:, :, r0 + 1:r0 + rcs + 1, :]
            p2 = rz_s[:, :, r0 + 2:r0 + rcs + 2, :]
            xup = jnp.where(lo_b, p0, p1)
            xdn = jnp.where(lo_b, p1, p2)
            acc = None
            for ki, v in ((0, xup), (1, cen), (2, xdn)):
                vl = jnp.concatenate([v[..., 1:], v[..., :1]], -1)
                vr = jnp.concatenate([v[..., -1:], v[..., :-1]], -1)
                part = tr[ki] * vr + tc[ki] * v + tl[ki] * vl
                acc = part if acc is None else acc + part
            s1v = s1v + jnp.sum(acc, axis=2, keepdims=True)
            s2v = s2v + jnp.sum(acc * acc, axis=2, keepdims=True)
            y_ref[:, :, r0:r0 + rcs, :] = acc.astype(y_ref.dtype)

        s1 = jnp.sum(jnp.sum(s1v, axis=3, keepdims=True), axis=0,
                     keepdims=True)
        s2 = jnp.sum(jnp.sum(s2v, axis=3, keepdims=True), axis=0,
                     keepdims=True)
        mean = s1 * inv_n
        var = s2 * inv_n - mean * mean
        g = g_ref[...].astype(f32).reshape(1, CB, 1, W2)
        b = b_ref[...].astype(f32).reshape(1, CB, 1, W2)
        scale = g * jax.lax.rsqrt(var + eps)
        bias = b - mean * scale

        for r0 in range(0, Hh, rc):
            rcs = min(rc, Hh - r0)
            yv = y_ref[:, :, r0:r0 + rcs, :].astype(f32)
            y_ref[:, :, r0:r0 + rcs, :] = jnp.maximum(
                yv * scale + bias, 0.0).astype(y_ref.dtype)

    return body


def _dcm(x, gamma, beta, cb=8, rc=8, eps=1e-5):
    B, C, H, W = x.shape
    assert H % 2 == 0 and C % cb == 0
    Hh, W2 = H // 2, 2 * W
    xd = x.reshape(B, C, Hh, W2)
    gl = jnp.broadcast_to(gamma.astype(jnp.float32).reshape(C, 1, 1),
                          (C, 1, W2))
    bl = jnp.broadcast_to(beta.astype(jnp.float32).reshape(C, 1, 1),
                          (C, 1, W2))
    body = _make_body(B, cb, H, W, float(eps), rc)
    yd = pl.pallas_call(
        body,
        out_shape=jax.ShapeDtypeStruct((B, C, Hh, W2), x.dtype),
        grid=(C // cb,),
        in_specs=[
            pl.BlockSpec((B, cb, Hh, W2), lambda c: (0, c, 0, 0)),
            pl.BlockSpec((cb, 1, W2), lambda c: (c, 0, 0)),
            pl.BlockSpec((cb, 1, W2), lambda c: (c, 0, 0)),
        ],
        out_specs=pl.BlockSpec((B, cb, Hh, W2), lambda c: (0, c, 0, 0)),
        scratch_shapes=[pltpu.VMEM((B, cb, Hh + 2, W2), jnp.float32)],
        compiler_params=pltpu.CompilerParams(
            dimension_semantics=("parallel",),
            vmem_limit_bytes=48 << 20),
    )(xd, gl, bl)
    return yd.reshape(B, C, H, W)


def kernel(x, gamma, beta):
    return _dcm(x, gamma, beta)
